# baseline (speedup 1.0000x reference)
import functools

import jax
import jax.numpy as jnp
from jax import lax
from jax.experimental import pallas as pl
from jax.experimental.pallas import tpu as pltpu
from jax.experimental.pallas import tpu_sc as plsc

BATCH = 128
DIM = 2048
NBLK = DIM // 128
NUM_WORKERS = 32
B_PER_W = BATCH // NUM_WORKERS


def _sc_gather(t_pad, nw_tab, iw_tab):
    mesh = plsc.VectorSubcoreMesh(core_axis_name="c", subcore_axis_name="s")

    @functools.partial(
        pl.kernel,
        mesh=mesh,
        out_type=(
            jax.ShapeDtypeStruct((BATCH, NBLK, 128), jnp.float32),
            jax.ShapeDtypeStruct((BATCH, NBLK, 128), jnp.float32),
        ),
        scratch_types=[
            pltpu.VMEM((8,), jnp.int32),
            pltpu.VMEM((B_PER_W, NBLK, 128), jnp.float32),
            pltpu.VMEM((B_PER_W, NBLK, 128), jnp.float32),
            pltpu.SemaphoreType.DMA,
            pltpu.SemaphoreType.DMA,
        ],
    )
    def k(t_hbm, nw_hbm, iw_hbm, nw_out, iw_out, idx_v, nw_v, iw_v,
          sem_g, sem_o):
        wid = lax.axis_index("s") * 2 + lax.axis_index("c")
        base = wid * B_PER_W

        pltpu.sync_copy(t_hbm.at[wid], idx_v)
        idx4 = idx_v.at[pl.ds(0, B_PER_W)]
        g_nw = pltpu.async_copy(nw_hbm.at[idx4], nw_v, sem_g)
        g_iw = pltpu.async_copy(iw_hbm.at[idx4], iw_v, sem_g)
        g_nw.wait()
        o_nw = pltpu.async_copy(nw_v, nw_out.at[pl.ds(base, B_PER_W)], sem_o)
        g_iw.wait()
        o_iw = pltpu.async_copy(iw_v, iw_out.at[pl.ds(base, B_PER_W)], sem_o)
        o_nw.wait()
        o_iw.wait()

    return k(t_pad, nw_tab.reshape(-1, NBLK, 128),
             iw_tab.reshape(-1, NBLK, 128))


def _tc_blend(x0_b, nz_b, nw_rows, iw_rows):
    grid = 8
    rows = BATCH // grid

    def body(x0_ref, nz_ref, nw_ref, iw_ref, out_ref):
        nw = nw_ref[...][:, :, None, :]
        iw = iw_ref[...][:, :, None, :]
        out_ref[...] = iw * x0_ref[...] + nw * nz_ref[...]

    bs4 = pl.BlockSpec((rows, NBLK, 2, 128), lambda i: (i, 0, 0, 0))
    bs3 = pl.BlockSpec((rows, NBLK, 128), lambda i: (i, 0, 0))
    return pl.pallas_call(
        body,
        grid=(grid,),
        in_specs=[bs4, bs4, bs3, bs3],
        out_specs=bs4,
        out_shape=jax.ShapeDtypeStruct((BATCH, NBLK, 2, 128), jnp.float32),
    )(x0_b, nz_b, nw_rows, iw_rows)


def kernel(x_0, t, task_id, noise, noise_weights, info_weights):
    del task_id
    t_pad = jnp.pad(t.astype(jnp.int32).reshape(NUM_WORKERS, B_PER_W),
                    ((0, 0), (0, 8 - B_PER_W)))
    x0_b = x_0.reshape(BATCH, NBLK, 128, 2).transpose(0, 1, 3, 2)
    nz_b = noise.reshape(BATCH, NBLK, 128, 2).transpose(0, 1, 3, 2)
    nw_rows, iw_rows = _sc_gather(t_pad, noise_weights, info_weights)
    out = _tc_blend(x0_b, nz_b, nw_rows, iw_rows)
    return out.transpose(0, 1, 3, 2).reshape(BATCH, DIM, 2)


# speedup vs baseline: 1.3108x; 1.3108x over previous
import functools

import jax
import jax.numpy as jnp
from jax import lax
from jax.experimental import pallas as pl
from jax.experimental.pallas import tpu as pltpu
from jax.experimental.pallas import tpu_sc as plsc

BATCH = 128
DIM = 2048
NBLK = DIM // 128
NSUB = 2 * NBLK
NUM_WORKERS = 32
B_PER_W = BATCH // NUM_WORKERS
LANES = 16


def _sc_blend(t_pad, nw_tab, iw_tab, x0_b, nz_b):
    mesh = plsc.VectorSubcoreMesh(core_axis_name="c", subcore_axis_name="s")

    @functools.partial(
        pl.kernel,
        mesh=mesh,
        out_type=jax.ShapeDtypeStruct((BATCH, NSUB, 128), jnp.float32),
        scratch_types=[
            pltpu.VMEM((B_PER_W,), jnp.int32),
            pltpu.VMEM((B_PER_W, DIM), jnp.float32),
            pltpu.VMEM((B_PER_W, DIM), jnp.float32),
            pltpu.VMEM((B_PER_W, NSUB, 128), jnp.float32),
            pltpu.VMEM((B_PER_W, NSUB, 128), jnp.float32),
            pltpu.VMEM((B_PER_W, NSUB, 128), jnp.float32),
            pltpu.SemaphoreType.DMA,
            pltpu.SemaphoreType.DMA,
            pltpu.SemaphoreType.DMA,
            pltpu.SemaphoreType.DMA,
            pltpu.SemaphoreType.DMA,
            pltpu.SemaphoreType.DMA,
        ],
    )
    def k(t_hbm, nw_hbm, iw_hbm, x0_hbm, nz_hbm, out_hbm,
          idx_v, nw_v, iw_v, x0_v, nz_v, out_v,
          sem_w, sem_r0, sem_r1, sem_r2, sem_r3, sem_o):
        wid = lax.axis_index("s") * 2 + lax.axis_index("c")
        base = wid * B_PER_W
        sem_r = (sem_r0, sem_r1, sem_r2, sem_r3)

        row_in = []
        for b in range(B_PER_W):
            h1 = pltpu.async_copy(x0_hbm.at[base + b], x0_v.at[b], sem_r[b])
            h2 = pltpu.async_copy(nz_hbm.at[base + b], nz_v.at[b], sem_r[b])
            row_in.append((h1, h2))

        pltpu.sync_copy(t_hbm.at[wid], idx_v)
        g_nw = pltpu.async_copy(nw_hbm.at[idx_v], nw_v, sem_w)
        g_iw = pltpu.async_copy(iw_hbm.at[idx_v], iw_v, sem_w)
        g_nw.wait()
        g_iw.wait()

        row_out = []
        for b in range(B_PER_W):
            h1, h2 = row_in[b]
            h1.wait()
            h2.wait()

            @plsc.parallel_loop(0, DIM // LANES, unroll=1)
            def chunk(kk, b=b):
                woff = kk * LANES
                j16 = (kk & 7) * LANES
                s0 = (kk >> 3) * 2
                nw16 = nw_v[b, pl.ds(woff, LANES)]
                iw16 = iw_v[b, pl.ds(woff, LANES)]
                for dc in (0, 1):
                    x0c = x0_v[b, s0 + dc, pl.ds(j16, LANES)]
                    nzc = nz_v[b, s0 + dc, pl.ds(j16, LANES)]
                    out_v[b, s0 + dc, pl.ds(j16, LANES)] = (
                        iw16 * x0c + nw16 * nzc)

            row_out.append(
                pltpu.async_copy(out_v.at[b], out_hbm.at[base + b], sem_o))

        for h in row_out:
            h.wait()

    return k(t_pad, nw_tab, iw_tab, x0_b, nz_b)


def kernel(x_0, t, task_id, noise, noise_weights, info_weights):
    del task_id
    t_pad = t.astype(jnp.int32).reshape(NUM_WORKERS, B_PER_W)
    x0_b = x_0.reshape(BATCH, NBLK, 128, 2).transpose(0, 1, 3, 2).reshape(
        BATCH, NSUB, 128)
    nz_b = noise.reshape(BATCH, NBLK, 128, 2).transpose(0, 1, 3, 2).reshape(
        BATCH, NSUB, 128)
    out = _sc_blend(t_pad, noise_weights, info_weights, x0_b, nz_b)
    return out.reshape(BATCH, NBLK, 2, 128).transpose(0, 1, 3, 2).reshape(
        BATCH, DIM, 2)
